# initial kernel scaffold (unmeasured)
import jax
import jax.numpy as jnp
from jax import lax
from jax.experimental import pallas as pl
from jax.experimental.pallas import tpu as pltpu


def kernel(
    x,
):
    def body(*refs):
        pass

    out_shape = jax.ShapeDtypeStruct(..., jnp.float32)
    return pl.pallas_call(body, out_shape=out_shape)(...)


# baseline (device time: 787353 ns/iter reference)
import jax
import jax.numpy as jnp
from jax import lax
from jax.experimental import pallas as pl
from jax.experimental.pallas import tpu as pltpu

M = 32768
N = 1024
QROWS = M // 4
HALF = M // 2


def kernel(x):
    xi = lax.axis_index("x")
    yi = lax.axis_index("y")
    zi = lax.axis_index("z")
    q = 2 * zi + xi
    xq = lax.dynamic_slice(x, (q * QROWS, 0), (QROWS, N)).astype(jnp.bfloat16)

    def body(xq_ref, out_ref, p1_ref, red_ref, sems):
        xi = lax.axis_index("x")
        yi = lax.axis_index("y")
        zi = lax.axis_index("z")
        q = 2 * zi + xi

        bsem = pltpu.get_barrier_semaphore()
        for nbr in ((xi, 1 - yi, zi), (1 - xi, yi, zi), (xi, yi, 1 - zi)):
            pl.semaphore_signal(
                bsem, inc=1, device_id=nbr, device_id_type=pl.DeviceIdType.MESH
            )
        pl.semaphore_wait(bsem, 3)

        p1 = pltpu.make_async_remote_copy(
            src_ref=xq_ref,
            dst_ref=p1_ref,
            send_sem=sems.at[0],
            recv_sem=sems.at[1],
            device_id=(xi, 1 - yi, zi),
            device_id_type=pl.DeviceIdType.MESH,
        )
        p1.start()
        p1.wait()

        red_ref[...] = xq_ref[...] + p1_ref[...]

        loc = pltpu.make_async_copy(
            red_ref, out_ref.at[pl.ds(q * QROWS, QROWS)], sems.at[6]
        )
        loc.start()

        p2 = pltpu.make_async_remote_copy(
            src_ref=red_ref,
            dst_ref=out_ref.at[pl.ds(q * QROWS, QROWS)],
            send_sem=sems.at[2],
            recv_sem=sems.at[3],
            device_id=(1 - xi, yi, zi),
            device_id_type=pl.DeviceIdType.MESH,
        )
        p2.start()
        p2.wait()
        loc.wait()

        p3 = pltpu.make_async_remote_copy(
            src_ref=out_ref.at[pl.ds(zi * HALF, HALF)],
            dst_ref=out_ref.at[pl.ds(zi * HALF, HALF)],
            send_sem=sems.at[4],
            recv_sem=sems.at[5],
            device_id=(xi, yi, 1 - zi),
            device_id_type=pl.DeviceIdType.MESH,
        )
        p3.start()
        p3.wait()

    return pl.pallas_call(
        body,
        out_shape=jax.ShapeDtypeStruct((M, N), jnp.bfloat16),
        in_specs=[pl.BlockSpec(memory_space=pltpu.VMEM)],
        out_specs=pl.BlockSpec(memory_space=pltpu.MemorySpace.HBM),
        scratch_shapes=[
            pltpu.VMEM((QROWS, N), jnp.bfloat16),
            pltpu.VMEM((QROWS, N), jnp.bfloat16),
            pltpu.SemaphoreType.DMA((7,)),
        ],
        compiler_params=pltpu.CompilerParams(collective_id=0),
    )(xq)


# device time: 446328 ns/iter; 1.7641x vs baseline; 1.7641x over previous
import jax
import jax.numpy as jnp
from jax import lax
from jax.experimental import pallas as pl
from jax.experimental.pallas import tpu as pltpu

M = 32768
N = 1024
QROWS = M // 4
C = 8
CH = QROWS // C


def kernel(x):
    xi = lax.axis_index("x")
    yi = lax.axis_index("y")
    zi = lax.axis_index("z")
    q = 2 * zi + xi
    xq = lax.dynamic_slice(x, (q * QROWS, 0), (QROWS, N)).astype(jnp.bfloat16)

    def body(xq_ref, out_ref, p1_ref, red_ref, p1s, p1r, p2s, p2r, locs, p3s, p3r):
        xi = lax.axis_index("x")
        yi = lax.axis_index("y")
        zi = lax.axis_index("z")
        q = 2 * zi + xi
        q2 = 2 * zi + (1 - xi)
        ynbr = (xi, 1 - yi, zi)
        xnbr = (1 - xi, yi, zi)
        znbr = (xi, yi, 1 - zi)

        bsem = pltpu.get_barrier_semaphore()
        for nbr in (ynbr, xnbr, znbr):
            pl.semaphore_signal(
                bsem, inc=1, device_id=nbr, device_id_type=pl.DeviceIdType.MESH
            )
        pl.semaphore_wait(bsem, 3)

        p1 = [
            pltpu.make_async_remote_copy(
                src_ref=xq_ref.at[pl.ds(c * CH, CH)],
                dst_ref=p1_ref.at[pl.ds(c * CH, CH)],
                send_sem=p1s.at[c],
                recv_sem=p1r.at[c],
                device_id=ynbr,
                device_id_type=pl.DeviceIdType.MESH,
            )
            for c in range(C)
        ]
        for c in range(C):
            p1[c].start()

        p2 = []
        p3q = []
        loc = []
        for c in range(C):
            p1[c].wait()
            sl = pl.ds(c * CH, CH)
            red_ref[sl] = xq_ref[sl] + p1_ref[sl]
            osl = pl.ds(q * QROWS + c * CH, CH)
            lc = pltpu.make_async_copy(red_ref.at[sl], out_ref.at[osl], locs.at[c])
            lc.start()
            loc.append(lc)
            p2c = pltpu.make_async_remote_copy(
                src_ref=red_ref.at[sl],
                dst_ref=out_ref.at[osl],
                send_sem=p2s.at[c],
                recv_sem=p2r.at[c],
                device_id=xnbr,
                device_id_type=pl.DeviceIdType.MESH,
            )
            p2c.start()
            p2.append(p2c)
            p3c = pltpu.make_async_remote_copy(
                src_ref=red_ref.at[sl],
                dst_ref=out_ref.at[osl],
                send_sem=p3s.at[c],
                recv_sem=p3r.at[c],
                device_id=znbr,
                device_id_type=pl.DeviceIdType.MESH,
            )
            p3c.start()
            p3q.append(p3c)

        p3q2 = []
        for c in range(C):
            p2[c].wait()
            osl2 = pl.ds(q2 * QROWS + c * CH, CH)
            p3c = pltpu.make_async_remote_copy(
                src_ref=out_ref.at[osl2],
                dst_ref=out_ref.at[osl2],
                send_sem=p3s.at[C + c],
                recv_sem=p3r.at[C + c],
                device_id=znbr,
                device_id_type=pl.DeviceIdType.MESH,
            )
            p3c.start()
            p3q2.append(p3c)

        for c in range(C):
            loc[c].wait()
            p3q[c].wait()
            p3q2[c].wait()

    return pl.pallas_call(
        body,
        out_shape=jax.ShapeDtypeStruct((M, N), jnp.bfloat16),
        in_specs=[pl.BlockSpec(memory_space=pltpu.VMEM)],
        out_specs=pl.BlockSpec(memory_space=pltpu.MemorySpace.HBM),
        scratch_shapes=[
            pltpu.VMEM((QROWS, N), jnp.bfloat16),
            pltpu.VMEM((QROWS, N), jnp.bfloat16),
            pltpu.SemaphoreType.DMA((C,)),
            pltpu.SemaphoreType.DMA((C,)),
            pltpu.SemaphoreType.DMA((C,)),
            pltpu.SemaphoreType.DMA((C,)),
            pltpu.SemaphoreType.DMA((C,)),
            pltpu.SemaphoreType.DMA((2 * C,)),
            pltpu.SemaphoreType.DMA((2 * C,)),
        ],
        compiler_params=pltpu.CompilerParams(collective_id=0),
    )(xq)


# device time: 383243 ns/iter; 2.0544x vs baseline; 1.1646x over previous
import jax
import jax.numpy as jnp
from jax import lax
from jax.experimental import pallas as pl
from jax.experimental.pallas import tpu as pltpu

M = 32768
N = 1024
QROWS = M // 4
C = 8
CH = QROWS // C
CA = C // 2


def kernel(x):
    xi = lax.axis_index("x")
    yi = lax.axis_index("y")
    zi = lax.axis_index("z")
    q = 2 * ((yi + zi) % 2) + (xi + zi) % 2
    qy = q ^ 2
    xq = lax.dynamic_slice(x, (q * QROWS, 0), (QROWS, N)).astype(jnp.bfloat16)
    xs = lax.dynamic_slice(x, (qy * QROWS, 0), (QROWS, N)).astype(jnp.bfloat16)

    def body(
        xq_ref, xs_ref, out_ref, p1_ref, red_ref,
        p1s, p1r, gxs, gxr, gzs, gzr, gyds, gydr, fwds, fwdr, locs,
    ):
        xi = lax.axis_index("x")
        yi = lax.axis_index("y")
        zi = lax.axis_index("z")
        q = 2 * ((yi + zi) % 2) + (xi + zi) % 2
        qz = q ^ 3
        ynbr = (xi, 1 - yi, zi)
        xnbr = (1 - xi, yi, zi)
        znbr = (xi, yi, 1 - zi)

        bsem = pltpu.get_barrier_semaphore()
        for nbr in (ynbr, xnbr, znbr):
            pl.semaphore_signal(
                bsem, inc=1, device_id=nbr, device_id_type=pl.DeviceIdType.MESH
            )
        pl.semaphore_wait(bsem, 3)

        p1 = [
            pltpu.make_async_remote_copy(
                src_ref=xs_ref.at[pl.ds(c * CH, CH)],
                dst_ref=p1_ref.at[pl.ds(c * CH, CH)],
                send_sem=p1s.at[c],
                recv_sem=p1r.at[c],
                device_id=ynbr,
                device_id_type=pl.DeviceIdType.MESH,
            )
            for c in range(C)
        ]
        for c in range(C):
            p1[c].start()

        gx = []
        gz = []
        gyd = []
        loc = []
        for c in range(C):
            p1[c].wait()
            sl = pl.ds(c * CH, CH)
            red_ref[sl] = xq_ref[sl] + p1_ref[sl]
            osl = pl.ds(q * QROWS + c * CH, CH)
            lc = pltpu.make_async_copy(red_ref.at[sl], out_ref.at[osl], locs.at[c])
            lc.start()
            loc.append(lc)
            gxc = pltpu.make_async_remote_copy(
                src_ref=red_ref.at[sl],
                dst_ref=out_ref.at[osl],
                send_sem=gxs.at[c],
                recv_sem=gxr.at[c],
                device_id=xnbr,
                device_id_type=pl.DeviceIdType.MESH,
            )
            gxc.start()
            gx.append(gxc)
            gzc = pltpu.make_async_remote_copy(
                src_ref=red_ref.at[sl],
                dst_ref=out_ref.at[osl],
                send_sem=gzs.at[c],
                recv_sem=gzr.at[c],
                device_id=znbr,
                device_id_type=pl.DeviceIdType.MESH,
            )
            gzc.start()
            gz.append(gzc)
            if c < CA:
                gydc = pltpu.make_async_remote_copy(
                    src_ref=red_ref.at[sl],
                    dst_ref=out_ref.at[osl],
                    send_sem=gyds.at[c],
                    recv_sem=gydr.at[c],
                    device_id=ynbr,
                    device_id_type=pl.DeviceIdType.MESH,
                )
                gydc.start()
                gyd.append(gydc)

        fwd = []
        for k in range(C - CA):
            c = CA + k
            gz[c].wait()
            osl = pl.ds(qz * QROWS + c * CH, CH)
            fc = pltpu.make_async_remote_copy(
                src_ref=out_ref.at[osl],
                dst_ref=out_ref.at[osl],
                send_sem=fwds.at[k],
                recv_sem=fwdr.at[k],
                device_id=xnbr,
                device_id_type=pl.DeviceIdType.MESH,
            )
            fc.start()
            fwd.append(fc)

        for c in range(C):
            loc[c].wait()
            gx[c].wait()
            if c < CA:
                gz[c].wait()
                gyd[c].wait()
            else:
                fwd[c - CA].wait()

    return pl.pallas_call(
        body,
        out_shape=jax.ShapeDtypeStruct((M, N), jnp.bfloat16),
        in_specs=[
            pl.BlockSpec(memory_space=pltpu.VMEM),
            pl.BlockSpec(memory_space=pltpu.MemorySpace.HBM),
        ],
        out_specs=pl.BlockSpec(memory_space=pltpu.MemorySpace.HBM),
        scratch_shapes=[
            pltpu.VMEM((QROWS, N), jnp.bfloat16),
            pltpu.VMEM((QROWS, N), jnp.bfloat16),
            pltpu.SemaphoreType.DMA((C,)),
            pltpu.SemaphoreType.DMA((C,)),
            pltpu.SemaphoreType.DMA((C,)),
            pltpu.SemaphoreType.DMA((C,)),
            pltpu.SemaphoreType.DMA((C,)),
            pltpu.SemaphoreType.DMA((C,)),
            pltpu.SemaphoreType.DMA((CA,)),
            pltpu.SemaphoreType.DMA((CA,)),
            pltpu.SemaphoreType.DMA((C - CA,)),
            pltpu.SemaphoreType.DMA((C - CA,)),
            pltpu.SemaphoreType.DMA((C,)),
        ],
        compiler_params=pltpu.CompilerParams(
            collective_id=0, vmem_limit_bytes=64 * 1024 * 1024
        ),
    )(xq, xs)


# device time: 301159 ns/iter; 2.6144x vs baseline; 1.2726x over previous
import jax
import jax.numpy as jnp
from jax import lax
from jax.experimental import pallas as pl
from jax.experimental.pallas import tpu as pltpu

M = 32768
N = 1024
QROWS = M // 4
C = 16
CH = QROWS // C
FD = 6
FX = 5
FZ = C - FD - FX


def kernel(x):
    def body(
        x_ref, out_ref, xs_ref, p1_ref, red_ref, s1_ref, s2_ref,
        xsd_sem, xqd_sem, p1s, p1r, gxs, gxr, gzs, gzr,
        gyds, gydr, fxs, fxr, fzs, fzr, loc_sem,
    ):
        xi = lax.axis_index("x")
        yi = lax.axis_index("y")
        zi = lax.axis_index("z")
        q = 2 * ((yi + zi) % 2) + (xi + zi) % 2
        qx = q ^ 1
        qy = q ^ 2
        qz = q ^ 3
        ynbr = (xi, 1 - yi, zi)
        xnbr = (1 - xi, yi, zi)
        znbr = (xi, yi, 1 - zi)

        bsem = pltpu.get_barrier_semaphore()
        for nbr in (ynbr, xnbr, znbr):
            pl.semaphore_signal(
                bsem, inc=1, device_id=nbr, device_id_type=pl.DeviceIdType.MESH
            )
        pl.semaphore_wait(bsem, 3)

        xsd = [
            pltpu.make_async_copy(
                x_ref.at[pl.ds(qy * QROWS + c * CH, CH)],
                s1_ref.at[c % 2],
                xsd_sem.at[c],
            )
            for c in range(C)
        ]
        xqd = [
            pltpu.make_async_copy(
                x_ref.at[pl.ds(q * QROWS + c * CH, CH)],
                s2_ref.at[c % 2],
                xqd_sem.at[c],
            )
            for c in range(C)
        ]
        p1 = [
            pltpu.make_async_remote_copy(
                src_ref=xs_ref.at[pl.ds(c * CH, CH)],
                dst_ref=p1_ref.at[pl.ds(c * CH, CH)],
                send_sem=p1s.at[c],
                recv_sem=p1r.at[c],
                device_id=ynbr,
                device_id_type=pl.DeviceIdType.MESH,
            )
            for c in range(C)
        ]

        xsd[0].start()
        xsd[1].start()
        xqd[0].start()
        xqd[1].start()

        for c in range(C):
            xsd[c].wait()
            xs_ref[pl.ds(c * CH, CH)] = s1_ref[c % 2].astype(jnp.bfloat16)
            if c + 2 < C:
                xsd[c + 2].start()
            p1[c].start()

        gx = []
        gz = []
        gyd = []
        for c in range(C):
            xqd[c].wait()
            p1[c].wait()
            sl = pl.ds(c * CH, CH)
            red_ref[sl] = s2_ref[c % 2].astype(jnp.bfloat16) + p1_ref[sl]
            if c + 2 < C:
                xqd[c + 2].start()
            osl = pl.ds(q * QROWS + c * CH, CH)
            gxc = pltpu.make_async_remote_copy(
                src_ref=red_ref.at[sl],
                dst_ref=out_ref.at[osl],
                send_sem=gxs.at[c],
                recv_sem=gxr.at[c],
                device_id=xnbr,
                device_id_type=pl.DeviceIdType.MESH,
            )
            gxc.start()
            gx.append(gxc)
            gzc = pltpu.make_async_remote_copy(
                src_ref=red_ref.at[sl],
                dst_ref=out_ref.at[osl],
                send_sem=gzs.at[c],
                recv_sem=gzr.at[c],
                device_id=znbr,
                device_id_type=pl.DeviceIdType.MESH,
            )
            gzc.start()
            gz.append(gzc)
            if c < FD:
                gydc = pltpu.make_async_remote_copy(
                    src_ref=red_ref.at[sl],
                    dst_ref=out_ref.at[osl],
                    send_sem=gyds.at[c],
                    recv_sem=gydr.at[c],
                    device_id=ynbr,
                    device_id_type=pl.DeviceIdType.MESH,
                )
                gydc.start()
                gyd.append(gydc)

        loc = pltpu.make_async_copy(
            red_ref, out_ref.at[pl.ds(q * QROWS, QROWS)], loc_sem
        )
        loc.start()

        fx = []
        for k in range(FX):
            c = FD + k
            gz[c].wait()
            osl = pl.ds(qz * QROWS + c * CH, CH)
            fc = pltpu.make_async_remote_copy(
                src_ref=out_ref.at[osl],
                dst_ref=out_ref.at[osl],
                send_sem=fxs.at[k],
                recv_sem=fxr.at[k],
                device_id=xnbr,
                device_id_type=pl.DeviceIdType.MESH,
            )
            fc.start()
            fx.append(fc)

        fz = []
        for j in range(FZ):
            c = FD + FX + j
            gx[c].wait()
            osl = pl.ds(qx * QROWS + c * CH, CH)
            fc = pltpu.make_async_remote_copy(
                src_ref=out_ref.at[osl],
                dst_ref=out_ref.at[osl],
                send_sem=fzs.at[j],
                recv_sem=fzr.at[j],
                device_id=znbr,
                device_id_type=pl.DeviceIdType.MESH,
            )
            fc.start()
            fz.append(fc)

        for c in range(C):
            if c < FD + FX:
                gx[c].wait()
            if not (FD <= c < FD + FX):
                gz[c].wait()
            if c < FD:
                gyd[c].wait()
        for k in range(FX):
            fx[k].wait()
        for j in range(FZ):
            fz[j].wait()
        loc.wait()

    return pl.pallas_call(
        body,
        out_shape=jax.ShapeDtypeStruct((M, N), jnp.bfloat16),
        in_specs=[pl.BlockSpec(memory_space=pltpu.MemorySpace.HBM)],
        out_specs=pl.BlockSpec(memory_space=pltpu.MemorySpace.HBM),
        scratch_shapes=[
            pltpu.VMEM((QROWS, N), jnp.bfloat16),
            pltpu.VMEM((QROWS, N), jnp.bfloat16),
            pltpu.VMEM((QROWS, N), jnp.bfloat16),
            pltpu.VMEM((2, CH, N), jnp.float32),
            pltpu.VMEM((2, CH, N), jnp.float32),
            pltpu.SemaphoreType.DMA((C,)),
            pltpu.SemaphoreType.DMA((C,)),
            pltpu.SemaphoreType.DMA((C,)),
            pltpu.SemaphoreType.DMA((C,)),
            pltpu.SemaphoreType.DMA((C,)),
            pltpu.SemaphoreType.DMA((C,)),
            pltpu.SemaphoreType.DMA((C,)),
            pltpu.SemaphoreType.DMA((C,)),
            pltpu.SemaphoreType.DMA((FD,)),
            pltpu.SemaphoreType.DMA((FD,)),
            pltpu.SemaphoreType.DMA((FX,)),
            pltpu.SemaphoreType.DMA((FX,)),
            pltpu.SemaphoreType.DMA((FZ,)),
            pltpu.SemaphoreType.DMA((FZ,)),
            pltpu.SemaphoreType.DMA,
        ],
        compiler_params=pltpu.CompilerParams(
            collective_id=0, vmem_limit_bytes=64 * 1024 * 1024
        ),
    )(x)
